# Initial kernel scaffold; baseline (speedup 1.0000x reference)
import jax
import jax.numpy as jnp
from jax.experimental import pallas as pl


def kernel(x_nchw, w_hwio, bias, prelu_a):
    raise NotImplementedError("write your pallas kernel here")


# speedup vs baseline: 1.0723x; 1.0723x over previous
import functools

import jax
import jax.numpy as jnp
from jax.experimental import pallas as pl
from jax.experimental.pallas import tpu as pltpu


def _conv_shuffle_kernel(a_ref, x_ref, w_ref, b_ref, o_ref, lhs_ref):
    TH = o_ref.shape[1]
    r = o_ref.shape[2]
    W = o_ref.shape[3]
    rco = o_ref.shape[4]
    cin = x_ref.shape[3]
    k9 = lhs_ref.shape[2]

    t = pl.program_id(1)
    row0 = pl.multiple_of(t * TH, TH)

    for kw in range(3):
        slab = x_ref[0, pl.ds(row0, TH + 2), kw:kw + W, :]
        for kh in range(3):
            tap = kh * 3 + kw
            lhs_ref[:, :, tap * cin:(tap + 1) * cin] = slab[kh:kh + TH]

    lhs = lhs_ref[...].reshape(TH * W, k9)
    acc = jnp.dot(lhs, w_ref[...], preferred_element_type=jnp.float32)
    acc = acc + b_ref[...]
    a = a_ref[0]
    acc = jnp.where(acc >= 0.0, acc, a * acc)

    y = acc.reshape(TH, W, r * rco)
    for i in range(r):
        o_ref[0, :, i, :, :] = y[:, :, i * rco:(i + 1) * rco]


@functools.partial(jax.jit, static_argnames=("r",))
def _run(x_nchw, w_hwio, bias, prelu_a, r=2):
    N, Cin, H, W = x_nchw.shape
    Cout = w_hwio.shape[-1]
    Co = Cout // (r * r)
    rco = r * Co
    K9 = 9 * Cin

    TH = 32
    while H % TH:
        TH //= 2
    n_bands = H // TH

    x_nhwc = jnp.transpose(x_nchw, (0, 2, 3, 1)).astype(jnp.bfloat16)
    x_pad = jnp.pad(x_nhwc, ((0, 0), (1, 1), (1, 1), (0, 0)))

    w2 = (w_hwio.reshape(K9, Co, r, r).transpose(0, 2, 3, 1)
          .reshape(K9, Cout).astype(jnp.bfloat16))
    b2 = bias.reshape(Co, r, r).transpose(1, 2, 0).reshape(1, Cout)
    b2 = b2.astype(jnp.float32)
    a_arr = jnp.asarray(prelu_a, dtype=jnp.float32).reshape(1)

    grid = (N, n_bands)
    cost = pl.CostEstimate(
        flops=2 * N * H * W * K9 * Cout,
        transcendentals=0,
        bytes_accessed=int(x_pad.size * 2 + w2.size * 2 + b2.size * 4
                           + N * H * r * W * rco * 4))

    out5 = pl.pallas_call(
        _conv_shuffle_kernel,
        out_shape=jax.ShapeDtypeStruct((N, H, r, W, rco), jnp.float32),
        grid=grid,
        in_specs=[
            pl.BlockSpec(memory_space=pltpu.SMEM),
            pl.BlockSpec((1, H + 2, W + 2, Cin), lambda n, t: (n, 0, 0, 0)),
            pl.BlockSpec((K9, Cout), lambda n, t: (0, 0)),
            pl.BlockSpec((1, Cout), lambda n, t: (0, 0)),
        ],
        out_specs=pl.BlockSpec((1, TH, r, W, rco), lambda n, t: (n, t, 0, 0, 0)),
        scratch_shapes=[pltpu.VMEM((TH, W, K9), jnp.bfloat16)],
        compiler_params=pltpu.CompilerParams(
            dimension_semantics=("parallel", "parallel"),
            vmem_limit_bytes=48 * 1024 * 1024),
        cost_estimate=cost,
    )(a_arr, x_pad, w2, b2)

    out_nhwc = out5.reshape(N, H * r, W * r, Co)
    return jnp.transpose(out_nhwc, (0, 3, 1, 2))


def kernel(x_nchw, w_hwio, bias, prelu_a):
    return _run(x_nchw, w_hwio, bias, prelu_a, r=2)


# speedup vs baseline: 1.1381x; 1.0614x over previous
import functools

import jax
import jax.numpy as jnp
from jax.experimental import pallas as pl
from jax.experimental.pallas import tpu as pltpu


def _conv_shuffle_kernel(a_ref, x_ref, w_ref, b_ref, o_ref, lhs_ref):
    TH = o_ref.shape[1]
    r = o_ref.shape[2]
    W = o_ref.shape[3]
    rco = o_ref.shape[4]
    cin = x_ref.shape[3]
    k9 = lhs_ref.shape[2]

    t = pl.program_id(1)
    row0 = pl.multiple_of(t * TH, TH)

    slab = x_ref[0, pl.ds(row0, TH + 2), :, :]
    zcol = jnp.zeros((TH + 2, 1, cin), jnp.bfloat16)
    shifted = (
        jnp.concatenate([zcol, slab[:, :W - 1, :]], axis=1),
        slab,
        jnp.concatenate([slab[:, 1:, :], zcol], axis=1),
    )
    for kw in range(3):
        s = shifted[kw]
        for kh in range(3):
            tap = kh * 3 + kw
            lhs_ref[:, :, tap * cin:(tap + 1) * cin] = s[kh:kh + TH]

    lhs = lhs_ref[...].reshape(TH * W, k9)
    acc = jnp.dot(lhs, w_ref[...], preferred_element_type=jnp.float32)
    acc = acc + b_ref[...]
    a = a_ref[0]
    acc = jnp.where(acc >= 0.0, acc, a * acc)

    y = acc.reshape(TH, W, r * rco).astype(jnp.bfloat16)
    for i in range(r):
        o_ref[0, :, i, :, :] = y[:, :, i * rco:(i + 1) * rco]


@functools.partial(jax.jit, static_argnames=("r",))
def _run(x_nchw, w_hwio, bias, prelu_a, r=2):
    N, Cin, H, W = x_nchw.shape
    Cout = w_hwio.shape[-1]
    Co = Cout // (r * r)
    rco = r * Co
    K9 = 9 * Cin

    TH = 32
    while H % TH:
        TH //= 2
    n_bands = H // TH

    x_nhwc = jnp.transpose(x_nchw, (0, 2, 3, 1)).astype(jnp.bfloat16)
    x_pad = jnp.pad(x_nhwc, ((0, 0), (1, 1), (0, 0), (0, 0)))

    w2 = (w_hwio.reshape(K9, Co, r, r).transpose(0, 2, 3, 1)
          .reshape(K9, Cout).astype(jnp.bfloat16))
    b2 = bias.reshape(Co, r, r).transpose(1, 2, 0).reshape(1, Cout)
    b2 = b2.astype(jnp.float32)
    a_arr = jnp.asarray(prelu_a, dtype=jnp.float32).reshape(1)

    grid = (N, n_bands)
    cost = pl.CostEstimate(
        flops=2 * N * H * W * K9 * Cout,
        transcendentals=0,
        bytes_accessed=int(x_pad.size * 2 + w2.size * 2 + b2.size * 4
                           + N * H * r * W * rco * 2))

    out5 = pl.pallas_call(
        _conv_shuffle_kernel,
        out_shape=jax.ShapeDtypeStruct((N, H, r, W, rco), jnp.bfloat16),
        grid=grid,
        in_specs=[
            pl.BlockSpec(memory_space=pltpu.SMEM),
            pl.BlockSpec((1, H + 2, W, Cin), lambda n, t: (n, 0, 0, 0)),
            pl.BlockSpec((K9, Cout), lambda n, t: (0, 0)),
            pl.BlockSpec((1, Cout), lambda n, t: (0, 0)),
        ],
        out_specs=pl.BlockSpec((1, TH, r, W, rco), lambda n, t: (n, t, 0, 0, 0)),
        scratch_shapes=[pltpu.VMEM((TH, W, K9), jnp.bfloat16)],
        compiler_params=pltpu.CompilerParams(
            dimension_semantics=("parallel", "parallel"),
            vmem_limit_bytes=48 * 1024 * 1024),
        cost_estimate=cost,
    )(a_arr, x_pad, w2, b2)

    out_nhwc = out5.reshape(N, H * r, W * r, Co)
    return jnp.transpose(out_nhwc, (0, 3, 1, 2)).astype(jnp.float32)


def kernel(x_nchw, w_hwio, bias, prelu_a):
    return _run(x_nchw, w_hwio, bias, prelu_a, r=2)


# speedup vs baseline: 1.3632x; 1.1978x over previous
import functools

import jax
import jax.numpy as jnp
from jax.experimental import pallas as pl
from jax.experimental.pallas import tpu as pltpu


def _fused_kernel(a_ref, x_ref, w_ref, b_ref, o_ref, xt_ref, lhs_ref, acc_ref):
    W, cin = xt_ref.shape[1], xt_ref.shape[2]
    TH = lhs_ref.shape[0]
    k9 = lhs_ref.shape[2]
    cout = acc_ref.shape[1]
    co = cout // 4

    t = pl.program_id(1)
    r0 = pl.multiple_of(t * TH, TH)
    a = a_ref[0]

    for c8 in range(TH // 8 + 1):
        blk = x_ref[0, :, pl.ds(r0 + 8 * c8, 8), :]
        t1 = jnp.transpose(blk, (1, 0, 2))
        xt_ref[8 * c8:8 * c8 + 8] = jnp.transpose(t1, (0, 2, 1)).astype(jnp.bfloat16)

    zcol = jnp.zeros((TH + 2, 1, cin), jnp.bfloat16)
    slab = xt_ref[0:TH + 2]
    shifted = (
        jnp.concatenate([zcol, slab[:, :W - 1, :]], axis=1),
        slab,
        jnp.concatenate([slab[:, 1:, :], zcol], axis=1),
    )
    for kw in range(3):
        s = shifted[kw]
        for kh in range(3):
            tap = kh * 3 + kw
            lhs_ref[:, :, tap * cin:(tap + 1) * cin] = s[kh:kh + TH]

    lhs = lhs_ref[...].reshape(TH * W, k9)
    acc_ref[...] = jnp.dot(lhs, w_ref[...], preferred_element_type=jnp.float32)

    ilv = jax.lax.broadcasted_iota(jnp.int32, (co, 128), 1)
    ilv = (ilv % 2) * 64 + (ilv // 2)
    for g in range(TH // 4):
        rows = []
        for h4 in range(4):
            hh = 4 * g + h4
            row = acc_ref[hh * W:(hh + 1) * W, :] + b_ref[...]
            row = jnp.where(row >= 0.0, row, a * row)
            th_t = jnp.transpose(row)
            for i in range(2):
                b0 = th_t[i * 2 * co:i * 2 * co + co]
                b1 = th_t[i * 2 * co + co:(i + 1) * 2 * co]
                d0 = jnp.concatenate([b0[:, :64], b1[:, :64]], axis=1)
                d1 = jnp.concatenate([b0[:, 64:], b1[:, 64:]], axis=1)
                g0 = jnp.take_along_axis(d0, ilv, axis=1)
                g1 = jnp.take_along_axis(d1, ilv, axis=1)
                rows.append(jnp.concatenate([g0, g1], axis=1))
        grp = jnp.transpose(jnp.stack(rows, axis=0), (1, 0, 2))
        o_ref[0, :, 8 * g:8 * g + 8, :] = grp


@functools.partial(jax.jit, static_argnames=("r",))
def _run(x_nchw, w_hwio, bias, prelu_a, r=2):
    N, Cin, H, W = x_nchw.shape
    Cout = w_hwio.shape[-1]
    Co = Cout // (r * r)
    K9 = 9 * Cin
    TH = 16
    while H % TH:
        TH //= 2
    n_bands = H // TH

    x_pad = jnp.pad(x_nchw, ((0, 0), (0, 0), (1, 7), (0, 0)))
    Hp = H + 8

    w2 = (w_hwio.reshape(K9, Co, r, r).transpose(0, 2, 3, 1)
          .reshape(K9, Cout).astype(jnp.bfloat16))
    b2 = bias.reshape(Co, r, r).transpose(1, 2, 0).reshape(1, Cout)
    b2 = b2.astype(jnp.float32)
    a_arr = jnp.asarray(prelu_a, dtype=jnp.float32).reshape(1)

    cost = pl.CostEstimate(
        flops=2 * N * H * W * K9 * Cout,
        transcendentals=0,
        bytes_accessed=int(x_pad.size * 4 + w2.size * 2 + b2.size * 4
                           + N * Co * 2 * H * 2 * W * 4))

    out = pl.pallas_call(
        _fused_kernel,
        out_shape=jax.ShapeDtypeStruct((N, Co, r * H, r * W), jnp.float32),
        grid=(N, n_bands),
        in_specs=[
            pl.BlockSpec(memory_space=pltpu.SMEM),
            pl.BlockSpec((1, Cin, Hp, W), lambda n, t: (n, 0, 0, 0)),
            pl.BlockSpec((K9, Cout), lambda n, t: (0, 0)),
            pl.BlockSpec((1, Cout), lambda n, t: (0, 0)),
        ],
        out_specs=pl.BlockSpec((1, Co, r * TH, r * W),
                               lambda n, t: (n, 0, t, 0)),
        scratch_shapes=[
            pltpu.VMEM((TH + 8, W, Cin), jnp.bfloat16),
            pltpu.VMEM((TH, W, K9), jnp.bfloat16),
            pltpu.VMEM((TH * W, Cout), jnp.float32),
        ],
        compiler_params=pltpu.CompilerParams(
            dimension_semantics=("parallel", "parallel"),
            vmem_limit_bytes=56 * 1024 * 1024),
        cost_estimate=cost,
    )(a_arr, x_pad, w2, b2)
    return out


def kernel(x_nchw, w_hwio, bias, prelu_a):
    return _run(x_nchw, w_hwio, bias, prelu_a, r=2)


# speedup vs baseline: 1.4075x; 1.0325x over previous
import functools

import jax
import jax.numpy as jnp
from jax.experimental import pallas as pl
from jax.experimental.pallas import tpu as pltpu


def _fused_kernel(a_ref, x_ref, w_ref, b_ref, o_ref, xt_ref, lhs_ref, acc_ref):
    W, cin = xt_ref.shape[1], xt_ref.shape[2]
    TH = lhs_ref.shape[0]
    k9 = lhs_ref.shape[2]
    cout = acc_ref.shape[1]
    co = cout // 4

    t = pl.program_id(1)
    r0 = pl.multiple_of(t * TH, TH)
    a = a_ref[0]

    for c8 in range(TH // 8 + 1):
        blk = x_ref[0, :, pl.ds(r0 + 8 * c8, 8), :]
        t1 = jnp.transpose(blk, (1, 0, 2))
        xt_ref[8 * c8:8 * c8 + 8] = jnp.transpose(t1, (0, 2, 1)).astype(jnp.bfloat16)

    zcol = jnp.zeros((TH + 2, 1, cin), jnp.bfloat16)
    slab = xt_ref[0:TH + 2]
    shifted = (
        jnp.concatenate([zcol, slab[:, :W - 1, :]], axis=1),
        slab,
        jnp.concatenate([slab[:, 1:, :], zcol], axis=1),
    )
    for kw in range(3):
        s = shifted[kw]
        for kh in range(3):
            tap = kh * 3 + kw
            lhs_ref[:, :, tap * cin:(tap + 1) * cin] = s[kh:kh + TH]

    lhs = lhs_ref[...].reshape(TH * W, k9)
    acc_ref[...] = jnp.dot(lhs, w_ref[...], preferred_element_type=jnp.float32)

    ilv = jax.lax.broadcasted_iota(jnp.int32, (co, 128), 1)
    ilv = (ilv % 2) * 64 + (ilv // 2)
    for g in range(TH // 4):
        rows = []
        for h4 in range(4):
            hh = 4 * g + h4
            row = acc_ref[hh * W:(hh + 1) * W, :] + b_ref[...]
            row = jnp.where(row >= 0.0, row, a * row)
            th_t = jnp.transpose(row)
            for i in range(2):
                b0 = th_t[i * 2 * co:i * 2 * co + co]
                b1 = th_t[i * 2 * co + co:(i + 1) * 2 * co]
                d0 = jnp.concatenate([b0[:, :64], b1[:, :64]], axis=1)
                d1 = jnp.concatenate([b0[:, 64:], b1[:, 64:]], axis=1)
                g0 = jnp.take_along_axis(d0, ilv, axis=1)
                g1 = jnp.take_along_axis(d1, ilv, axis=1)
                rows.append(jnp.concatenate([g0, g1], axis=1))
        grp = jnp.transpose(jnp.stack(rows, axis=0), (1, 0, 2))
        o_ref[0, :, 8 * g:8 * g + 8, :] = grp


@functools.partial(jax.jit, static_argnames=("r",))
def _run(x_nchw, w_hwio, bias, prelu_a, r=2):
    N, Cin, H, W = x_nchw.shape
    Cout = w_hwio.shape[-1]
    Co = Cout // (r * r)
    K9 = 9 * Cin
    TH = 32
    while H % TH:
        TH //= 2
    n_bands = H // TH

    x_pad = jnp.pad(x_nchw, ((0, 0), (0, 0), (1, 7), (0, 0)))
    Hp = H + 8

    w2 = (w_hwio.reshape(K9, Co, r, r).transpose(0, 2, 3, 1)
          .reshape(K9, Cout).astype(jnp.bfloat16))
    b2 = bias.reshape(Co, r, r).transpose(1, 2, 0).reshape(1, Cout)
    b2 = b2.astype(jnp.float32)
    a_arr = jnp.asarray(prelu_a, dtype=jnp.float32).reshape(1)

    cost = pl.CostEstimate(
        flops=2 * N * H * W * K9 * Cout,
        transcendentals=0,
        bytes_accessed=int(x_pad.size * 4 + w2.size * 2 + b2.size * 4
                           + N * Co * 2 * H * 2 * W * 4))

    out = pl.pallas_call(
        _fused_kernel,
        out_shape=jax.ShapeDtypeStruct((N, Co, r * H, r * W), jnp.float32),
        grid=(N, n_bands),
        in_specs=[
            pl.BlockSpec(memory_space=pltpu.SMEM),
            pl.BlockSpec((1, Cin, Hp, W), lambda n, t: (n, 0, 0, 0)),
            pl.BlockSpec((K9, Cout), lambda n, t: (0, 0)),
            pl.BlockSpec((1, Cout), lambda n, t: (0, 0)),
        ],
        out_specs=pl.BlockSpec((1, Co, r * TH, r * W),
                               lambda n, t: (n, 0, t, 0)),
        scratch_shapes=[
            pltpu.VMEM((TH + 8, W, Cin), jnp.bfloat16),
            pltpu.VMEM((TH, W, K9), jnp.bfloat16),
            pltpu.VMEM((TH * W, Cout), jnp.float32),
        ],
        compiler_params=pltpu.CompilerParams(
            dimension_semantics=("parallel", "arbitrary"),
            vmem_limit_bytes=56 * 1024 * 1024),
        cost_estimate=cost,
    )(a_arr, x_pad, w2, b2)
    return out


def kernel(x_nchw, w_hwio, bias, prelu_a):
    return _run(x_nchw, w_hwio, bias, prelu_a, r=2)


# speedup vs baseline: 1.4779x; 1.0500x over previous
import functools

import jax
import jax.numpy as jnp
from jax.experimental import pallas as pl
from jax.experimental.pallas import tpu as pltpu


def _fused_kernel(a_ref, x_ref, w_ref, b_ref, o_ref, xt_ref, lhs_ref, acc_ref):
    W, cin = xt_ref.shape[1], xt_ref.shape[2]
    TH = lhs_ref.shape[0]
    k9 = lhs_ref.shape[2]
    cout = acc_ref.shape[1]
    co = cout // 4

    t = pl.program_id(1)
    r0 = pl.multiple_of(t * TH, TH)
    a = a_ref[0]

    for c8 in range(TH // 8 + 1):
        blk = x_ref[0, :, pl.ds(r0 + 8 * c8, 8), :]
        t1 = jnp.transpose(blk, (1, 0, 2))
        xt_ref[8 * c8:8 * c8 + 8] = jnp.transpose(t1, (0, 2, 1)).astype(jnp.bfloat16)

    zcol = jnp.zeros((TH + 2, 1, cin), jnp.bfloat16)
    slab = xt_ref[0:TH + 2]
    shifted = (
        jnp.concatenate([zcol, slab[:, :W - 1, :]], axis=1),
        slab,
        jnp.concatenate([slab[:, 1:, :], zcol], axis=1),
    )
    for kw in range(3):
        s = shifted[kw]
        for kh in range(3):
            tap = kh * 3 + kw
            lhs_ref[:, :, tap * cin:(tap + 1) * cin] = s[kh:kh + TH]

    lhs = lhs_ref[...].reshape(TH * W, k9)
    acc = jnp.dot(lhs, w_ref[...], preferred_element_type=jnp.float32)
    acc = acc + b_ref[...]
    acc_ref[...] = jnp.where(acc >= 0.0, acc, a * acc).astype(jnp.bfloat16)

    ilv = jax.lax.broadcasted_iota(jnp.int32, (co, 128), 1)
    ilv = (ilv % 2) * 64 + (ilv // 2)
    for g in range(TH // 4):
        rows = []
        for h4 in range(4):
            hh = 4 * g + h4
            th_t = jnp.transpose(acc_ref[hh * W:(hh + 1) * W, :])
            for i in range(2):
                b0 = th_t[i * 2 * co:i * 2 * co + co]
                b1 = th_t[i * 2 * co + co:(i + 1) * 2 * co]
                d0 = jnp.concatenate([b0[:, :64], b1[:, :64]], axis=1)
                d1 = jnp.concatenate([b0[:, 64:], b1[:, 64:]], axis=1)
                g0 = jnp.take_along_axis(d0.astype(jnp.float32), ilv, axis=1)
                g1 = jnp.take_along_axis(d1.astype(jnp.float32), ilv, axis=1)
                rows.append(jnp.concatenate([g0, g1], axis=1))
        grp = jnp.transpose(jnp.stack(rows, axis=0), (1, 0, 2))
        o_ref[0, :, 8 * g:8 * g + 8, :] = grp


@functools.partial(jax.jit, static_argnames=("r",))
def _run(x_nchw, w_hwio, bias, prelu_a, r=2):
    N, Cin, H, W = x_nchw.shape
    Cout = w_hwio.shape[-1]
    Co = Cout // (r * r)
    K9 = 9 * Cin
    TH = 32
    while H % TH:
        TH //= 2
    n_bands = H // TH

    x_pad = jnp.pad(x_nchw, ((0, 0), (0, 0), (1, 7), (0, 0)))
    Hp = H + 8

    w2 = (w_hwio.reshape(K9, Co, r, r).transpose(0, 2, 3, 1)
          .reshape(K9, Cout).astype(jnp.bfloat16))
    b2 = bias.reshape(Co, r, r).transpose(1, 2, 0).reshape(1, Cout)
    b2 = b2.astype(jnp.float32)
    a_arr = jnp.asarray(prelu_a, dtype=jnp.float32).reshape(1)

    cost = pl.CostEstimate(
        flops=2 * N * H * W * K9 * Cout,
        transcendentals=0,
        bytes_accessed=int(x_pad.size * 4 + w2.size * 2 + b2.size * 4
                           + N * Co * 2 * H * 2 * W * 4))

    out = pl.pallas_call(
        _fused_kernel,
        out_shape=jax.ShapeDtypeStruct((N, Co, r * H, r * W), jnp.float32),
        grid=(N, n_bands),
        in_specs=[
            pl.BlockSpec(memory_space=pltpu.SMEM),
            pl.BlockSpec((1, Cin, Hp, W), lambda n, t: (n, 0, 0, 0)),
            pl.BlockSpec((K9, Cout), lambda n, t: (0, 0)),
            pl.BlockSpec((1, Cout), lambda n, t: (0, 0)),
        ],
        out_specs=pl.BlockSpec((1, Co, r * TH, r * W),
                               lambda n, t: (n, 0, t, 0)),
        scratch_shapes=[
            pltpu.VMEM((TH + 8, W, Cin), jnp.bfloat16),
            pltpu.VMEM((TH, W, K9), jnp.bfloat16),
            pltpu.VMEM((TH * W, Cout), jnp.bfloat16),
        ],
        compiler_params=pltpu.CompilerParams(
            dimension_semantics=("parallel", "arbitrary"),
            vmem_limit_bytes=56 * 1024 * 1024),
        cost_estimate=cost,
    )(a_arr, x_pad, w2, b2)
    return out


def kernel(x_nchw, w_hwio, bias, prelu_a):
    return _run(x_nchw, w_hwio, bias, prelu_a, r=2)


# speedup vs baseline: 1.4842x; 1.0043x over previous
import functools

import jax
import jax.numpy as jnp
from jax.experimental import pallas as pl
from jax.experimental.pallas import tpu as pltpu


def _fused_kernel(a_ref, x_ref, w_ref, b_ref, o_ref, xt_ref, lhs_ref, acc_ref):
    W, cin = xt_ref.shape[1], xt_ref.shape[2]
    TH = lhs_ref.shape[0]
    k9 = lhs_ref.shape[2]
    cout = acc_ref.shape[1]
    co = cout // 4

    t = pl.program_id(1)
    r0 = pl.multiple_of(t * TH, TH)
    a = a_ref[0]

    for c8 in range(TH // 8 + 1):
        blk = x_ref[0, :, pl.ds(r0 + 8 * c8, 8), :]
        t1 = jnp.transpose(blk, (1, 0, 2))
        xt_ref[8 * c8:8 * c8 + 8] = jnp.transpose(t1, (0, 2, 1))

    zcol = jnp.zeros((TH + 2, 1, cin), jnp.bfloat16)
    slab = xt_ref[0:TH + 2]
    shifted = (
        jnp.concatenate([zcol, slab[:, :W - 1, :]], axis=1),
        slab,
        jnp.concatenate([slab[:, 1:, :], zcol], axis=1),
    )
    for kw in range(3):
        s = shifted[kw]
        for kh in range(3):
            tap = kh * 3 + kw
            lhs_ref[:, :, tap * cin:(tap + 1) * cin] = s[kh:kh + TH]

    lhs = lhs_ref[...].reshape(TH * W, k9)
    acc = jnp.dot(lhs, w_ref[...], preferred_element_type=jnp.float32)
    acc = acc + b_ref[...]
    acc_ref[...] = jnp.where(acc >= 0.0, acc, a * acc).astype(jnp.bfloat16)

    ilv = jax.lax.broadcasted_iota(jnp.int32, (co, 128), 1)
    ilv = (ilv % 2) * 64 + (ilv // 2)
    for g in range(TH // 4):
        rows = []
        for h4 in range(4):
            hh = 4 * g + h4
            th_t = jnp.transpose(acc_ref[hh * W:(hh + 1) * W, :])
            for i in range(2):
                b0 = th_t[i * 2 * co:i * 2 * co + co]
                b1 = th_t[i * 2 * co + co:(i + 1) * 2 * co]
                d0 = jnp.concatenate([b0[:, :64], b1[:, :64]], axis=1)
                d1 = jnp.concatenate([b0[:, 64:], b1[:, 64:]], axis=1)
                g0 = jnp.take_along_axis(d0.astype(jnp.float32), ilv, axis=1)
                g1 = jnp.take_along_axis(d1.astype(jnp.float32), ilv, axis=1)
                rows.append(jnp.concatenate([g0, g1], axis=1))
        grp = jnp.transpose(jnp.stack(rows, axis=0), (1, 0, 2))
        o_ref[0, :, 8 * g:8 * g + 8, :] = grp


@functools.partial(jax.jit, static_argnames=("r",))
def _run(x_nchw, w_hwio, bias, prelu_a, r=2):
    N, Cin, H, W = x_nchw.shape
    Cout = w_hwio.shape[-1]
    Co = Cout // (r * r)
    K9 = 9 * Cin
    TH = 32
    while H % TH:
        TH //= 2
    n_bands = H // TH

    x_pad = jnp.pad(x_nchw, ((0, 0), (0, 0), (1, 7), (0, 0))).astype(jnp.bfloat16)
    Hp = H + 8

    w2 = (w_hwio.reshape(K9, Co, r, r).transpose(0, 2, 3, 1)
          .reshape(K9, Cout).astype(jnp.bfloat16))
    b2 = bias.reshape(Co, r, r).transpose(1, 2, 0).reshape(1, Cout)
    b2 = b2.astype(jnp.float32)
    a_arr = jnp.asarray(prelu_a, dtype=jnp.float32).reshape(1)

    cost = pl.CostEstimate(
        flops=2 * N * H * W * K9 * Cout,
        transcendentals=0,
        bytes_accessed=int(x_pad.size * 2 + w2.size * 2 + b2.size * 4
                           + N * Co * 2 * H * 2 * W * 4))

    out = pl.pallas_call(
        _fused_kernel,
        out_shape=jax.ShapeDtypeStruct((N, Co, r * H, r * W), jnp.float32),
        grid=(N, n_bands),
        in_specs=[
            pl.BlockSpec(memory_space=pltpu.SMEM),
            pl.BlockSpec((1, Cin, Hp, W), lambda n, t: (n, 0, 0, 0)),
            pl.BlockSpec((K9, Cout), lambda n, t: (0, 0)),
            pl.BlockSpec((1, Cout), lambda n, t: (0, 0)),
        ],
        out_specs=pl.BlockSpec((1, Co, r * TH, r * W),
                               lambda n, t: (n, 0, t, 0)),
        scratch_shapes=[
            pltpu.VMEM((TH + 8, W, Cin), jnp.bfloat16),
            pltpu.VMEM((TH, W, K9), jnp.bfloat16),
            pltpu.VMEM((TH * W, Cout), jnp.bfloat16),
        ],
        compiler_params=pltpu.CompilerParams(
            dimension_semantics=("parallel", "arbitrary"),
            vmem_limit_bytes=56 * 1024 * 1024),
        cost_estimate=cost,
    )(a_arr, x_pad, w2, b2)
    return out


def kernel(x_nchw, w_hwio, bias, prelu_a):
    return _run(x_nchw, w_hwio, bias, prelu_a, r=2)


# speedup vs baseline: 1.5228x; 1.0260x over previous
import functools

import jax
import jax.numpy as jnp
from jax.experimental import pallas as pl
from jax.experimental.pallas import tpu as pltpu


def _fused_kernel(a_ref, x_ref, w_ref, b_ref, o_ref, xt_ref, lhs_ref, acc_ref):
    W, cin = xt_ref.shape[1], xt_ref.shape[2]
    TH = lhs_ref.shape[0]
    k9 = lhs_ref.shape[2]
    cout = acc_ref.shape[1]
    co = cout // 4

    t = pl.program_id(1)
    r0 = pl.multiple_of(t * TH, TH)
    a = a_ref[0]

    for c8 in range(TH // 8 + 1):
        blk = x_ref[0, :, pl.ds(r0 + 8 * c8, 8), :]
        t1 = jnp.transpose(blk, (1, 0, 2))
        xt_ref[8 * c8:8 * c8 + 8] = jnp.transpose(t1, (0, 2, 1))

    zcol = jnp.zeros((TH + 2, 1, cin), jnp.bfloat16)
    slab = xt_ref[0:TH + 2]
    shifted = (
        jnp.concatenate([zcol, slab[:, :W - 1, :]], axis=1),
        slab,
        jnp.concatenate([slab[:, 1:, :], zcol], axis=1),
    )
    for kw in range(3):
        s = shifted[kw]
        for kh in range(3):
            tap = kh * 3 + kw
            lhs_ref[:, :, tap * cin:(tap + 1) * cin] = s[kh:kh + TH]

    lhs = lhs_ref[...].reshape(TH * W, k9)
    acc = jnp.dot(lhs, w_ref[...], preferred_element_type=jnp.float32)
    acc = acc + b_ref[...]
    acc_ref[...] = jnp.where(acc >= 0.0, acc, a * acc).astype(jnp.bfloat16)

    ilv = jax.lax.broadcasted_iota(jnp.int32, (co, 128), 1)
    ilv = (ilv % 2) * 64 + (ilv // 2)
    for g in range(TH // 4):
        rows = []
        for h4 in range(4):
            hh = 4 * g + h4
            th_t = jnp.transpose(acc_ref[hh * W:(hh + 1) * W, :])
            for i in range(2):
                b0 = th_t[i * 2 * co:i * 2 * co + co]
                b1 = th_t[i * 2 * co + co:(i + 1) * 2 * co]
                d0 = jnp.concatenate([b0[:, :64], b1[:, :64]], axis=1)
                d1 = jnp.concatenate([b0[:, 64:], b1[:, 64:]], axis=1)
                g0 = jnp.take_along_axis(d0.astype(jnp.float32), ilv, axis=1)
                g1 = jnp.take_along_axis(d1.astype(jnp.float32), ilv, axis=1)
                rows.append(jnp.concatenate([g0, g1], axis=1))
        grp = jnp.transpose(jnp.stack(rows, axis=0), (1, 0, 2))
        o_ref[0, :, 8 * g:8 * g + 8, :] = grp


@functools.partial(jax.jit, static_argnames=("r",))
def _run(x_nchw, w_hwio, bias, prelu_a, r=2):
    N, Cin, H, W = x_nchw.shape
    Cout = w_hwio.shape[-1]
    Co = Cout // (r * r)
    K9 = 9 * Cin
    TH = 64
    while H % TH:
        TH //= 2
    n_bands = H // TH

    x_pad = jnp.pad(x_nchw, ((0, 0), (0, 0), (1, 7), (0, 0))).astype(jnp.bfloat16)
    Hp = H + 8

    w2 = (w_hwio.reshape(K9, Co, r, r).transpose(0, 2, 3, 1)
          .reshape(K9, Cout).astype(jnp.bfloat16))
    b2 = bias.reshape(Co, r, r).transpose(1, 2, 0).reshape(1, Cout)
    b2 = b2.astype(jnp.float32)
    a_arr = jnp.asarray(prelu_a, dtype=jnp.float32).reshape(1)

    cost = pl.CostEstimate(
        flops=2 * N * H * W * K9 * Cout,
        transcendentals=0,
        bytes_accessed=int(x_pad.size * 2 + w2.size * 2 + b2.size * 4
                           + N * Co * 2 * H * 2 * W * 4))

    out = pl.pallas_call(
        _fused_kernel,
        out_shape=jax.ShapeDtypeStruct((N, Co, r * H, r * W), jnp.float32),
        grid=(N, n_bands),
        in_specs=[
            pl.BlockSpec(memory_space=pltpu.SMEM),
            pl.BlockSpec((1, Cin, Hp, W), lambda n, t: (n, 0, 0, 0)),
            pl.BlockSpec((K9, Cout), lambda n, t: (0, 0)),
            pl.BlockSpec((1, Cout), lambda n, t: (0, 0)),
        ],
        out_specs=pl.BlockSpec((1, Co, r * TH, r * W),
                               lambda n, t: (n, 0, t, 0)),
        scratch_shapes=[
            pltpu.VMEM((TH + 8, W, Cin), jnp.bfloat16),
            pltpu.VMEM((TH, W, K9), jnp.bfloat16),
            pltpu.VMEM((TH * W, Cout), jnp.bfloat16),
        ],
        compiler_params=pltpu.CompilerParams(
            dimension_semantics=("parallel", "arbitrary"),
            vmem_limit_bytes=56 * 1024 * 1024),
        cost_estimate=cost,
    )(a_arr, x_pad, w2, b2)
    return out


def kernel(x_nchw, w_hwio, bias, prelu_a):
    return _run(x_nchw, w_hwio, bias, prelu_a, r=2)
